# baseline (speedup 1.0000x reference)
import functools

import jax
import jax.numpy as jnp
from jax import lax
from jax.experimental import pallas as pl
from jax.experimental.pallas import tpu as pltpu
from jax.experimental.pallas import tpu_sc as plsc

H1 = W1 = 2544
H0 = W0 = 636
N = 2097152
HW1 = H1 * W1
HW0 = H0 * W0
EXT1 = 2 * HW1 + 2 * W1 + 8

_L = 16
_K = 1024
_G = _K // _L
_NCORN = 8


def _flat1(table):
    flat = table.reshape(2 * HW1)
    return jnp.concatenate([flat, flat[: EXT1 - 2 * HW1]])


def _pairs0(table):
    flat = table.reshape(HW0, 3)
    ext = jnp.concatenate([flat, flat[:1]], axis=0)
    return jnp.concatenate(
        [ext[:HW0], ext[1 : HW0 + 1], jnp.zeros((HW0, 2), jnp.float32)], axis=1
    )


def _buf_types():
    return (
        [
            pltpu.VMEM((_K,), jnp.float32),
            pltpu.VMEM((_K,), jnp.float32),
            pltpu.VMEM((_K,), jnp.int32),
            pltpu.VMEM((_K,), jnp.int32),
            pltpu.VMEM((_K,), jnp.float32),
            pltpu.VMEM((_K,), jnp.float32),
        ]
        + [pltpu.VMEM((_K,), jnp.int32) for _ in range(_NCORN)]
        + [pltpu.VMEM((_K,), jnp.float32) for _ in range(_NCORN)]
        + [
            pltpu.VMEM((_K, 8), jnp.float32),
            pltpu.VMEM((_K, 8), jnp.float32),
            pltpu.VMEM((3 * _K,), jnp.float32),
        ]
    )


@functools.cache
def _build():
    info = plsc.get_sparse_core_info()
    nc = info.num_cores
    nw = nc * info.num_subcores
    per_w = N // nw
    chunks = per_w // _K
    iters = chunks // 2

    mesh = plsc.VectorSubcoreMesh(core_axis_name="c", subcore_axis_name="s")

    @functools.partial(
        pl.kernel,
        mesh=mesh,
        out_type=jax.ShapeDtypeStruct((3 * N,), jnp.float32),
        compiler_params=pltpu.CompilerParams(
            needs_layout_passes=False, use_tc_tiling_on_sc=False
        ),
        scratch_types=_buf_types() + _buf_types() + [pltpu.SemaphoreType.DMA] * 4,
    )
    def grid_lookup(xu_hbm, xv_hbm, f1_hbm, p0_hbm, out_hbm, *refs):
        nb = len(_buf_types())
        bufs = (tuple(refs[0:nb]), tuple(refs[nb : 2 * nb]))
        sem_x, sem_g1, sem_g2, sem_o = refs[2 * nb : 2 * nb + 4]
        wid = lax.axis_index("s") * nc + lax.axis_index("c")
        base0 = wid * per_w

        def x_issue(b, ch):
            base = base0 + ch * _K
            pltpu.async_copy(xu_hbm.at[pl.ds(base, _K)], b[0], sem_x)
            pltpu.async_copy(xv_hbm.at[pl.ds(base, _K)], b[1], sem_x)

        def x_wait(b):
            pltpu.make_async_copy(xu_hbm.at[pl.ds(0, _K)], b[0], sem_x).wait()
            pltpu.make_async_copy(xv_hbm.at[pl.ds(0, _K)], b[1], sem_x).wait()

        def g1_issue(b):
            for k in range(_NCORN):
                pltpu.async_copy(f1_hbm.at[b[6 + k]], b[14 + k], sem_g1)

        def g1_wait(b):
            for k in range(_NCORN):
                pltpu.make_async_copy(f1_hbm.at[b[6 + k]], b[14 + k], sem_g1).wait()

        def g2_issue(b):
            pltpu.async_copy(p0_hbm.at[b[2]], b[22], sem_g2)
            pltpu.async_copy(p0_hbm.at[b[3]], b[23], sem_g2)

        def g2_wait(b):
            pltpu.make_async_copy(p0_hbm.at[b[2]], b[22], sem_g2).wait()
            pltpu.make_async_copy(p0_hbm.at[b[3]], b[23], sem_g2).wait()

        def o_issue(b, ch):
            base = base0 + ch * _K
            pltpu.async_copy(b[24], out_hbm.at[pl.ds(3 * base, 3 * _K)], sem_o)

        def o_wait(b):
            pltpu.make_async_copy(b[24], out_hbm.at[pl.ds(0, 3 * _K)], sem_o).wait()

        def phase1(b):
            def _g1b(g, carry):
                sl = pl.ds(g * _L, _L)
                u = jnp.clip(b[0][sl], 0.0, 1.0) * float(W1 - 1)
                v = jnp.clip(b[1][sl], 0.0, 1.0) * float(H1 - 1)
                u0 = u.astype(jnp.int32)
                v0 = v.astype(jnp.int32)
                b[4][sl] = u - u0.astype(jnp.float32)
                b[5][sl] = v - v0.astype(jnp.float32)
                base2 = (v0 * W1 + u0) * 2
                b[6][sl] = base2
                b[7][sl] = base2 + 1
                b[8][sl] = base2 + 2
                b[9][sl] = base2 + 3
                b[10][sl] = base2 + 2 * W1
                b[11][sl] = base2 + 2 * W1 + 1
                b[12][sl] = base2 + 2 * W1 + 2
                b[13][sl] = base2 + 2 * W1 + 3
                return carry

            lax.fori_loop(0, _G, _g1b, 0)

        def phase2(b):
            def _g2b(g, carry):
                sl = pl.ds(g * _L, _L)
                fu = b[4][sl]
                fv = b[5][sl]
                key = []
                for c in range(2):
                    t00 = b[14 + c][sl]
                    t01 = b[16 + c][sl]
                    t10 = b[18 + c][sl]
                    t11 = b[20 + c][sl]
                    top = t00 + fu * (t01 - t00)
                    bot = t10 + fu * (t11 - t10)
                    key.append(top + fv * (bot - top))
                u = jnp.clip(key[0], 0.0, 1.0) * float(W0 - 1)
                v = jnp.clip(key[1], 0.0, 1.0) * float(H0 - 1)
                u0 = u.astype(jnp.int32)
                v0 = v.astype(jnp.int32)
                b[4][sl] = u - u0.astype(jnp.float32)
                b[5][sl] = v - v0.astype(jnp.float32)
                v1 = jnp.minimum(v0 + 1, H0 - 1)
                b[2][sl] = v0 * W0 + u0
                b[3][sl] = v1 * W0 + u0
                return carry

            lax.fori_loop(0, _G, _g2b, 0)

        def phase3(b):
            def _g3b(g, carry):
                sl = pl.ds(g * _L, _L)
                pid = lax.iota(jnp.int32, _L) + g * _L
                pid3 = pid * 3
                fu = b[4][sl]
                fv = b[5][sl]
                for c in range(3):
                    t00 = plsc.load_gather(b[22], [pid, jnp.full((_L,), c, jnp.int32)])
                    t01 = plsc.load_gather(b[22], [pid, jnp.full((_L,), 3 + c, jnp.int32)])
                    t10 = plsc.load_gather(b[23], [pid, jnp.full((_L,), c, jnp.int32)])
                    t11 = plsc.load_gather(b[23], [pid, jnp.full((_L,), 3 + c, jnp.int32)])
                    top = t00 + fu * (t01 - t00)
                    bot = t10 + fu * (t11 - t10)
                    res = top + fv * (bot - top)
                    plsc.store_scatter(b[24], [pid3 + c], res)
                return carry

            lax.fori_loop(0, _G, _g3b, 0)

        A, B = bufs

        x_issue(A, 0)
        x_wait(A)
        phase1(A)
        g1_issue(A)
        x_issue(B, 1)

        def body(i, carry):
            ch0 = 2 * i
            g1_wait(A)
            phase2(A)
            g2_issue(A)
            x_wait(B)
            phase1(B)
            g1_issue(B)
            x_issue(A, jnp.minimum(ch0 + 2, chunks - 1))
            g2_wait(A)

            @pl.when(i > 0)
            def _():
                o_wait(A)

            phase3(A)
            o_issue(A, ch0)
            g1_wait(B)
            phase2(B)
            g2_issue(B)
            x_wait(A)
            phase1(A)
            g1_issue(A)
            x_issue(B, jnp.minimum(ch0 + 3, chunks - 1))
            g2_wait(B)

            @pl.when(i > 0)
            def _():
                o_wait(B)

            phase3(B)
            o_issue(B, ch0 + 1)
            return carry

        lax.fori_loop(0, iters, body, 0)

        g1_wait(A)
        x_wait(B)
        o_wait(A)
        o_wait(B)

    return grid_lookup


def kernel(x, grid1_table, grid0_table):
    xu = x[:, 0]
    xv = x[:, 1]
    out_flat = _build()(xu, xv, _flat1(grid1_table), _pairs0(grid0_table))
    return out_flat.reshape(N, 3)


# speedup vs baseline: 1.6656x; 1.6656x over previous
import functools

import jax
import jax.numpy as jnp
from jax import lax
from jax.experimental import pallas as pl
from jax.experimental.pallas import tpu as pltpu
from jax.experimental.pallas import tpu_sc as plsc

H1 = W1 = 2544
H0 = W0 = 636
N = 2097152
HW1 = H1 * W1
HW0 = H0 * W0

_L = 16
_K = 1024
_G = _K // _L


def _quad1(table):
    flat = table.reshape(HW1, 2)
    ext = jnp.concatenate([flat, flat[: W1 + 1]], axis=0)
    return jnp.concatenate(
        [ext[:HW1], ext[1 : HW1 + 1], ext[W1 : HW1 + W1], ext[W1 + 1 : HW1 + W1 + 1]],
        axis=1,
    )


def _pairs0(table):
    flat = table.reshape(HW0, 3)
    ext = jnp.concatenate([flat, flat[:1]], axis=0)
    return jnp.concatenate(
        [ext[:HW0], ext[1 : HW0 + 1], jnp.zeros((HW0, 2), jnp.float32)], axis=1
    )


def _buf_types():
    return [
        pltpu.VMEM((_K,), jnp.float32),
        pltpu.VMEM((_K,), jnp.float32),
        pltpu.VMEM((_K,), jnp.int32),
        pltpu.VMEM((_K,), jnp.int32),
        pltpu.VMEM((_K,), jnp.float32),
        pltpu.VMEM((_K,), jnp.float32),
        pltpu.VMEM((_K, 8), jnp.float32),
        pltpu.VMEM((_K, 8), jnp.float32),
        pltpu.VMEM((_K, 8), jnp.float32),
        pltpu.VMEM((3 * _K,), jnp.float32),
    ]


@functools.cache
def _build():
    info = plsc.get_sparse_core_info()
    nc = info.num_cores
    nw = nc * info.num_subcores
    per_w = N // nw
    chunks = per_w // _K
    iters = chunks // 2

    mesh = plsc.VectorSubcoreMesh(core_axis_name="c", subcore_axis_name="s")

    @functools.partial(
        pl.kernel,
        mesh=mesh,
        out_type=jax.ShapeDtypeStruct((3 * N,), jnp.float32),
        compiler_params=pltpu.CompilerParams(
            needs_layout_passes=False, use_tc_tiling_on_sc=False
        ),
        scratch_types=_buf_types() + _buf_types() + [pltpu.SemaphoreType.DMA] * 4,
    )
    def grid_lookup(xu_hbm, xv_hbm, q1_hbm, p0_hbm, out_hbm, *refs):
        bufs = (tuple(refs[0:10]), tuple(refs[10:20]))
        sem_x, sem_g1, sem_g2, sem_o = refs[20:24]
        wid = lax.axis_index("s") * nc + lax.axis_index("c")
        base0 = wid * per_w

        def x_issue(b, ch):
            base = base0 + ch * _K
            pltpu.async_copy(xu_hbm.at[pl.ds(base, _K)], b[0], sem_x)
            pltpu.async_copy(xv_hbm.at[pl.ds(base, _K)], b[1], sem_x)

        def x_wait(b):
            pltpu.make_async_copy(xu_hbm.at[pl.ds(0, _K)], b[0], sem_x).wait()
            pltpu.make_async_copy(xv_hbm.at[pl.ds(0, _K)], b[1], sem_x).wait()

        def g1_issue(b):
            pltpu.async_copy(q1_hbm.at[b[2]], b[6], sem_g1)

        def g1_wait(b):
            pltpu.make_async_copy(q1_hbm.at[b[2]], b[6], sem_g1).wait()

        def g2_issue(b):
            pltpu.async_copy(p0_hbm.at[b[2]], b[7], sem_g2)
            pltpu.async_copy(p0_hbm.at[b[3]], b[8], sem_g2)

        def g2_wait(b):
            pltpu.make_async_copy(p0_hbm.at[b[2]], b[7], sem_g2).wait()
            pltpu.make_async_copy(p0_hbm.at[b[3]], b[8], sem_g2).wait()

        def o_issue(b, ch):
            base = base0 + ch * _K
            pltpu.async_copy(b[9], out_hbm.at[pl.ds(3 * base, 3 * _K)], sem_o)

        def o_wait(b):
            pltpu.make_async_copy(b[9], out_hbm.at[pl.ds(0, 3 * _K)], sem_o).wait()

        def phase1(b):
            def _g1b(g, carry):
                sl = pl.ds(g * _L, _L)
                u = jnp.clip(b[0][sl], 0.0, 1.0) * float(W1 - 1)
                v = jnp.clip(b[1][sl], 0.0, 1.0) * float(H1 - 1)
                u0 = u.astype(jnp.int32)
                v0 = v.astype(jnp.int32)
                b[4][sl] = u - u0.astype(jnp.float32)
                b[5][sl] = v - v0.astype(jnp.float32)
                b[2][sl] = v0 * W1 + u0
                return carry

            lax.fori_loop(0, _G, _g1b, 0)

        def phase2(b):
            def _g2b(g, carry):
                sl = pl.ds(g * _L, _L)
                pid = lax.iota(jnp.int32, _L) + g * _L
                fu = b[4][sl]
                fv = b[5][sl]
                key = []
                for c in range(2):
                    t00 = plsc.load_gather(b[6], [pid, jnp.full((_L,), c, jnp.int32)])
                    t01 = plsc.load_gather(b[6], [pid, jnp.full((_L,), 2 + c, jnp.int32)])
                    t10 = plsc.load_gather(b[6], [pid, jnp.full((_L,), 4 + c, jnp.int32)])
                    t11 = plsc.load_gather(b[6], [pid, jnp.full((_L,), 6 + c, jnp.int32)])
                    top = t00 + fu * (t01 - t00)
                    bot = t10 + fu * (t11 - t10)
                    key.append(top + fv * (bot - top))
                u = jnp.clip(key[0], 0.0, 1.0) * float(W0 - 1)
                v = jnp.clip(key[1], 0.0, 1.0) * float(H0 - 1)
                u0 = u.astype(jnp.int32)
                v0 = v.astype(jnp.int32)
                b[4][sl] = u - u0.astype(jnp.float32)
                b[5][sl] = v - v0.astype(jnp.float32)
                v1 = jnp.minimum(v0 + 1, H0 - 1)
                b[2][sl] = v0 * W0 + u0
                b[3][sl] = v1 * W0 + u0
                return carry

            lax.fori_loop(0, _G, _g2b, 0)

        def phase3(b):
            def _g3b(g, carry):
                sl = pl.ds(g * _L, _L)
                pid = lax.iota(jnp.int32, _L) + g * _L
                pid3 = pid * 3
                fu = b[4][sl]
                fv = b[5][sl]
                for c in range(3):
                    t00 = plsc.load_gather(b[7], [pid, jnp.full((_L,), c, jnp.int32)])
                    t01 = plsc.load_gather(b[7], [pid, jnp.full((_L,), 3 + c, jnp.int32)])
                    t10 = plsc.load_gather(b[8], [pid, jnp.full((_L,), c, jnp.int32)])
                    t11 = plsc.load_gather(b[8], [pid, jnp.full((_L,), 3 + c, jnp.int32)])
                    top = t00 + fu * (t01 - t00)
                    bot = t10 + fu * (t11 - t10)
                    res = top + fv * (bot - top)
                    plsc.store_scatter(b[9], [pid3 + c], res)
                return carry

            lax.fori_loop(0, _G, _g3b, 0)

        A, B = bufs

        x_issue(A, 0)
        x_wait(A)
        phase1(A)
        g1_issue(A)
        x_issue(B, 1)

        def body(i, carry):
            ch0 = 2 * i
            g1_wait(A)
            phase2(A)
            g2_issue(A)
            x_wait(B)
            phase1(B)
            g1_issue(B)
            x_issue(A, jnp.minimum(ch0 + 2, chunks - 1))
            g2_wait(A)

            @pl.when(i > 0)
            def _():
                o_wait(A)

            phase3(A)
            o_issue(A, ch0)
            g1_wait(B)
            phase2(B)
            g2_issue(B)
            x_wait(A)
            phase1(A)
            g1_issue(A)
            x_issue(B, jnp.minimum(ch0 + 3, chunks - 1))
            g2_wait(B)

            @pl.when(i > 0)
            def _():
                o_wait(B)

            phase3(B)
            o_issue(B, ch0 + 1)
            return carry

        lax.fori_loop(0, iters, body, 0)

        g1_wait(A)
        x_wait(B)
        o_wait(A)
        o_wait(B)

    return grid_lookup


def kernel(x, grid1_table, grid0_table):
    xu = x[:, 0]
    xv = x[:, 1]
    out_flat = _build()(xu, xv, _quad1(grid1_table), _pairs0(grid0_table))
    return out_flat.reshape(N, 3)


# speedup vs baseline: 1.7630x; 1.0585x over previous
import functools

import jax
import jax.numpy as jnp
from jax import lax
from jax.experimental import pallas as pl
from jax.experimental.pallas import tpu as pltpu
from jax.experimental.pallas import tpu_sc as plsc

H1 = W1 = 2544
H0 = W0 = 636
N = 2097152
HW1 = H1 * W1
HW0 = H0 * W0

_L = 16
_K = 1024
_G = _K // _L


def _quad1(table):
    flat = table.reshape(HW1, 2)
    ext = jnp.concatenate([flat, flat[: W1 + 1]], axis=0)
    return jnp.concatenate(
        [ext[:HW1], ext[1 : HW1 + 1], ext[W1 : HW1 + W1], ext[W1 + 1 : HW1 + W1 + 1]],
        axis=1,
    )


def _pairs0(table):
    flat = table.reshape(HW0, 3)
    ext = jnp.concatenate([flat, flat[:1]], axis=0)
    return jnp.concatenate(
        [ext[:HW0], ext[1 : HW0 + 1], jnp.zeros((HW0, 2), jnp.float32)], axis=1
    )


def _buf_types():
    return [
        pltpu.VMEM((_K,), jnp.float32),
        pltpu.VMEM((_K,), jnp.float32),
        pltpu.VMEM((_K,), jnp.int32),
        pltpu.VMEM((_K,), jnp.int32),
        pltpu.VMEM((_K,), jnp.float32),
        pltpu.VMEM((_K,), jnp.float32),
        pltpu.VMEM((_K, 8), jnp.float32),
        pltpu.VMEM((_K, 8), jnp.float32),
        pltpu.VMEM((_K, 8), jnp.float32),
        pltpu.VMEM((_K, 3), jnp.float32),
    ]


@functools.cache
def _build():
    info = plsc.get_sparse_core_info()
    nc = info.num_cores
    nw = nc * info.num_subcores
    per_w = N // nw
    chunks = per_w // _K
    iters = chunks // 2

    mesh = plsc.VectorSubcoreMesh(core_axis_name="c", subcore_axis_name="s")

    @functools.partial(
        pl.kernel,
        mesh=mesh,
        out_type=jax.ShapeDtypeStruct((N, 3), jnp.float32),
        compiler_params=pltpu.CompilerParams(
            needs_layout_passes=False, use_tc_tiling_on_sc=False
        ),
        scratch_types=_buf_types() + _buf_types() + [pltpu.SemaphoreType.DMA] * 4,
    )
    def grid_lookup(xu_hbm, xv_hbm, q1_hbm, p0_hbm, out_hbm, *refs):
        bufs = (tuple(refs[0:10]), tuple(refs[10:20]))
        sem_x, sem_g1, sem_g2, sem_o = refs[20:24]
        wid = lax.axis_index("s") * nc + lax.axis_index("c")
        base0 = wid * per_w

        def x_issue(b, ch):
            base = base0 + ch * _K
            pltpu.async_copy(xu_hbm.at[pl.ds(base, _K)], b[0], sem_x)
            pltpu.async_copy(xv_hbm.at[pl.ds(base, _K)], b[1], sem_x)

        def x_wait(b):
            pltpu.make_async_copy(xu_hbm.at[pl.ds(0, _K)], b[0], sem_x).wait()
            pltpu.make_async_copy(xv_hbm.at[pl.ds(0, _K)], b[1], sem_x).wait()

        def g1_issue(b):
            pltpu.async_copy(q1_hbm.at[b[2]], b[6], sem_g1)

        def g1_wait(b):
            pltpu.make_async_copy(q1_hbm.at[b[2]], b[6], sem_g1).wait()

        def g2_issue(b):
            pltpu.async_copy(p0_hbm.at[b[2]], b[7], sem_g2)
            pltpu.async_copy(p0_hbm.at[b[3]], b[8], sem_g2)

        def g2_wait(b):
            pltpu.make_async_copy(p0_hbm.at[b[2]], b[7], sem_g2).wait()
            pltpu.make_async_copy(p0_hbm.at[b[3]], b[8], sem_g2).wait()

        def o_issue(b, ch):
            base = base0 + ch * _K
            pltpu.async_copy(b[9], out_hbm.at[pl.ds(base, _K)], sem_o)

        def o_wait(b):
            pltpu.make_async_copy(b[9], out_hbm.at[pl.ds(0, _K)], sem_o).wait()

        def phase1(b):
            def _g1b(g, carry):
                sl = pl.ds(g * _L, _L)
                u = jnp.clip(b[0][sl], 0.0, 1.0) * float(W1 - 1)
                v = jnp.clip(b[1][sl], 0.0, 1.0) * float(H1 - 1)
                u0 = u.astype(jnp.int32)
                v0 = v.astype(jnp.int32)
                b[4][sl] = u - u0.astype(jnp.float32)
                b[5][sl] = v - v0.astype(jnp.float32)
                b[2][sl] = v0 * W1 + u0
                return carry

            lax.fori_loop(0, _G, _g1b, 0)

        def phase2(b):
            def _g2b(g, carry):
                sl = pl.ds(g * _L, _L)
                pid = lax.iota(jnp.int32, _L) + g * _L
                fu = b[4][sl]
                fv = b[5][sl]
                key = []
                for c in range(2):
                    t00 = plsc.load_gather(b[6], [pid, jnp.full((_L,), c, jnp.int32)])
                    t01 = plsc.load_gather(b[6], [pid, jnp.full((_L,), 2 + c, jnp.int32)])
                    t10 = plsc.load_gather(b[6], [pid, jnp.full((_L,), 4 + c, jnp.int32)])
                    t11 = plsc.load_gather(b[6], [pid, jnp.full((_L,), 6 + c, jnp.int32)])
                    top = t00 + fu * (t01 - t00)
                    bot = t10 + fu * (t11 - t10)
                    key.append(top + fv * (bot - top))
                u = jnp.clip(key[0], 0.0, 1.0) * float(W0 - 1)
                v = jnp.clip(key[1], 0.0, 1.0) * float(H0 - 1)
                u0 = u.astype(jnp.int32)
                v0 = v.astype(jnp.int32)
                b[4][sl] = u - u0.astype(jnp.float32)
                b[5][sl] = v - v0.astype(jnp.float32)
                v1 = jnp.minimum(v0 + 1, H0 - 1)
                b[2][sl] = v0 * W0 + u0
                b[3][sl] = v1 * W0 + u0
                return carry

            lax.fori_loop(0, _G, _g2b, 0)

        def phase3(b):
            def _g3b(g, carry):
                sl = pl.ds(g * _L, _L)
                pid = lax.iota(jnp.int32, _L) + g * _L
                fu = b[4][sl]
                fv = b[5][sl]
                for c in range(3):
                    t00 = plsc.load_gather(b[7], [pid, jnp.full((_L,), c, jnp.int32)])
                    t01 = plsc.load_gather(b[7], [pid, jnp.full((_L,), 3 + c, jnp.int32)])
                    t10 = plsc.load_gather(b[8], [pid, jnp.full((_L,), c, jnp.int32)])
                    t11 = plsc.load_gather(b[8], [pid, jnp.full((_L,), 3 + c, jnp.int32)])
                    top = t00 + fu * (t01 - t00)
                    bot = t10 + fu * (t11 - t10)
                    res = top + fv * (bot - top)
                    plsc.store_scatter(b[9], [pid, jnp.full((_L,), c, jnp.int32)], res)
                return carry

            lax.fori_loop(0, _G, _g3b, 0)

        A, B = bufs

        x_issue(A, 0)
        x_wait(A)
        phase1(A)
        g1_issue(A)
        x_issue(B, 1)

        def body(i, carry):
            ch0 = 2 * i
            g1_wait(A)
            phase2(A)
            g2_issue(A)
            x_wait(B)
            phase1(B)
            g1_issue(B)
            x_issue(A, jnp.minimum(ch0 + 2, chunks - 1))
            g2_wait(A)

            @pl.when(i > 0)
            def _():
                o_wait(A)

            phase3(A)
            o_issue(A, ch0)
            g1_wait(B)
            phase2(B)
            g2_issue(B)
            x_wait(A)
            phase1(A)
            g1_issue(A)
            x_issue(B, jnp.minimum(ch0 + 3, chunks - 1))
            g2_wait(B)

            @pl.when(i > 0)
            def _():
                o_wait(B)

            phase3(B)
            o_issue(B, ch0 + 1)
            return carry

        lax.fori_loop(0, iters, body, 0)

        g1_wait(A)
        x_wait(B)
        o_wait(A)
        o_wait(B)

    return grid_lookup


def kernel(x, grid1_table, grid0_table):
    xu = x[:, 0]
    xv = x[:, 1]
    return _build()(xu, xv, _quad1(grid1_table), _pairs0(grid0_table))


# speedup vs baseline: 2.3122x; 1.3115x over previous
import functools

import jax
import jax.numpy as jnp
from jax import lax
from jax.experimental import pallas as pl
from jax.experimental.pallas import tpu as pltpu
from jax.experimental.pallas import tpu_sc as plsc

H1 = W1 = 2544
H0 = W0 = 636
N = 2097152
HW1 = H1 * W1
HW0 = H0 * W0

_L = 16
_K = 1024
_G = _K // _L


def _quad1(table):
    q01 = jnp.roll(table, -1, axis=1)
    q10 = jnp.roll(table, -1, axis=0)
    q11 = jnp.roll(q10, -1, axis=1)
    return jnp.concatenate([table, q01, q10, q11], axis=2).reshape(HW1, 8)


def _pairs0(table):
    flat = table.reshape(HW0, 3)
    ext = jnp.concatenate([flat, flat[:1]], axis=0)
    return jnp.concatenate(
        [ext[:HW0], ext[1 : HW0 + 1], jnp.zeros((HW0, 2), jnp.float32)], axis=1
    )


def _buf_types():
    return [
        pltpu.VMEM((_K,), jnp.float32),
        pltpu.VMEM((_K,), jnp.float32),
        pltpu.VMEM((_K,), jnp.int32),
        pltpu.VMEM((_K,), jnp.int32),
        pltpu.VMEM((_K,), jnp.float32),
        pltpu.VMEM((_K,), jnp.float32),
        pltpu.VMEM((_K, 8), jnp.float32),
        pltpu.VMEM((_K, 8), jnp.float32),
        pltpu.VMEM((_K, 8), jnp.float32),
        pltpu.VMEM((_K, 3), jnp.float32),
    ]


@functools.cache
def _build():
    info = plsc.get_sparse_core_info()
    nc = info.num_cores
    nw = nc * info.num_subcores
    per_w = N // nw
    chunks = per_w // _K
    iters = chunks // 2

    mesh = plsc.VectorSubcoreMesh(core_axis_name="c", subcore_axis_name="s")

    @functools.partial(
        pl.kernel,
        mesh=mesh,
        out_type=jax.ShapeDtypeStruct((N, 3), jnp.float32),
        compiler_params=pltpu.CompilerParams(
            needs_layout_passes=False, use_tc_tiling_on_sc=False
        ),
        scratch_types=_buf_types() + _buf_types() + [pltpu.SemaphoreType.DMA] * 4,
    )
    def grid_lookup(xu_hbm, xv_hbm, q1_hbm, p0_hbm, out_hbm, *refs):
        bufs = (tuple(refs[0:10]), tuple(refs[10:20]))
        sem_x, sem_g1, sem_g2, sem_o = refs[20:24]
        wid = lax.axis_index("s") * nc + lax.axis_index("c")
        base0 = wid * per_w

        def x_issue(b, ch):
            base = base0 + ch * _K
            pltpu.async_copy(xu_hbm.at[pl.ds(base, _K)], b[0], sem_x)
            pltpu.async_copy(xv_hbm.at[pl.ds(base, _K)], b[1], sem_x)

        def x_wait(b):
            pltpu.make_async_copy(xu_hbm.at[pl.ds(0, _K)], b[0], sem_x).wait()
            pltpu.make_async_copy(xv_hbm.at[pl.ds(0, _K)], b[1], sem_x).wait()

        def g1_issue(b):
            pltpu.async_copy(q1_hbm.at[b[2]], b[6], sem_g1)

        def g1_wait(b):
            pltpu.make_async_copy(q1_hbm.at[b[2]], b[6], sem_g1).wait()

        def g2_issue(b):
            pltpu.async_copy(p0_hbm.at[b[2]], b[7], sem_g2)
            pltpu.async_copy(p0_hbm.at[b[3]], b[8], sem_g2)

        def g2_wait(b):
            pltpu.make_async_copy(p0_hbm.at[b[2]], b[7], sem_g2).wait()
            pltpu.make_async_copy(p0_hbm.at[b[3]], b[8], sem_g2).wait()

        def o_issue(b, ch):
            base = base0 + ch * _K
            pltpu.async_copy(b[9], out_hbm.at[pl.ds(base, _K)], sem_o)

        def o_wait(b):
            pltpu.make_async_copy(b[9], out_hbm.at[pl.ds(0, _K)], sem_o).wait()

        def phase1(b):
            def _g1b(g, carry):
                sl = pl.ds(g * _L, _L)
                u = jnp.clip(b[0][sl], 0.0, 1.0) * float(W1 - 1)
                v = jnp.clip(b[1][sl], 0.0, 1.0) * float(H1 - 1)
                u0 = u.astype(jnp.int32)
                v0 = v.astype(jnp.int32)
                b[4][sl] = u - u0.astype(jnp.float32)
                b[5][sl] = v - v0.astype(jnp.float32)
                b[2][sl] = v0 * W1 + u0
                return carry

            lax.fori_loop(0, _G, _g1b, 0)

        def phase2(b):
            def _g2b(g, carry):
                sl = pl.ds(g * _L, _L)
                pid = lax.iota(jnp.int32, _L) + g * _L
                fu = b[4][sl]
                fv = b[5][sl]
                key = []
                for c in range(2):
                    t00 = plsc.load_gather(b[6], [pid, jnp.full((_L,), c, jnp.int32)])
                    t01 = plsc.load_gather(b[6], [pid, jnp.full((_L,), 2 + c, jnp.int32)])
                    t10 = plsc.load_gather(b[6], [pid, jnp.full((_L,), 4 + c, jnp.int32)])
                    t11 = plsc.load_gather(b[6], [pid, jnp.full((_L,), 6 + c, jnp.int32)])
                    top = t00 + fu * (t01 - t00)
                    bot = t10 + fu * (t11 - t10)
                    key.append(top + fv * (bot - top))
                u = jnp.clip(key[0], 0.0, 1.0) * float(W0 - 1)
                v = jnp.clip(key[1], 0.0, 1.0) * float(H0 - 1)
                u0 = u.astype(jnp.int32)
                v0 = v.astype(jnp.int32)
                b[4][sl] = u - u0.astype(jnp.float32)
                b[5][sl] = v - v0.astype(jnp.float32)
                v1 = jnp.minimum(v0 + 1, H0 - 1)
                b[2][sl] = v0 * W0 + u0
                b[3][sl] = v1 * W0 + u0
                return carry

            lax.fori_loop(0, _G, _g2b, 0)

        def phase3(b):
            def _g3b(g, carry):
                sl = pl.ds(g * _L, _L)
                pid = lax.iota(jnp.int32, _L) + g * _L
                fu = b[4][sl]
                fv = b[5][sl]
                for c in range(3):
                    t00 = plsc.load_gather(b[7], [pid, jnp.full((_L,), c, jnp.int32)])
                    t01 = plsc.load_gather(b[7], [pid, jnp.full((_L,), 3 + c, jnp.int32)])
                    t10 = plsc.load_gather(b[8], [pid, jnp.full((_L,), c, jnp.int32)])
                    t11 = plsc.load_gather(b[8], [pid, jnp.full((_L,), 3 + c, jnp.int32)])
                    top = t00 + fu * (t01 - t00)
                    bot = t10 + fu * (t11 - t10)
                    res = top + fv * (bot - top)
                    plsc.store_scatter(b[9], [pid, jnp.full((_L,), c, jnp.int32)], res)
                return carry

            lax.fori_loop(0, _G, _g3b, 0)

        A, B = bufs

        x_issue(A, 0)
        x_wait(A)
        phase1(A)
        g1_issue(A)
        x_issue(B, 1)

        def body(i, carry):
            ch0 = 2 * i
            g1_wait(A)
            phase2(A)
            g2_issue(A)
            x_wait(B)
            phase1(B)
            g1_issue(B)
            x_issue(A, jnp.minimum(ch0 + 2, chunks - 1))
            g2_wait(A)

            @pl.when(i > 0)
            def _():
                o_wait(A)

            phase3(A)
            o_issue(A, ch0)
            g1_wait(B)
            phase2(B)
            g2_issue(B)
            x_wait(A)
            phase1(A)
            g1_issue(A)
            x_issue(B, jnp.minimum(ch0 + 3, chunks - 1))
            g2_wait(B)

            @pl.when(i > 0)
            def _():
                o_wait(B)

            phase3(B)
            o_issue(B, ch0 + 1)
            return carry

        lax.fori_loop(0, iters, body, 0)

        g1_wait(A)
        x_wait(B)
        o_wait(A)
        o_wait(B)

    return grid_lookup


def kernel(x, grid1_table, grid0_table):
    xu = x[:, 0]
    xv = x[:, 1]
    return _build()(xu, xv, _quad1(grid1_table), _pairs0(grid0_table))
